# baseline (speedup 1.0000x reference)
import functools

import jax
import jax.numpy as jnp
from jax import lax
from jax.experimental import pallas as pl
from jax.experimental.pallas import tpu as pltpu
from jax.experimental.pallas import tpu_sc as plsc

_N = 10000
_E = 320000
_D = 128
_CH = 128
_NW = 32
_NCHUNK = _E // _CH
_CPW = 80
_EPAD_ROWS = _NW * _CPW
_ACC_ROWS = 10016
_ZOFF = 624
_ZWIN = 656
_NPH = 2
_CPP = _CPW // _NPH

_mesh = plsc.VectorSubcoreMesh(
    core_axis_name="c", subcore_axis_name="s", num_cores=2, num_subcores=16
)


_NBUF = 2


_UNROLL = 8


def _emit_pass(gather, vals_hbm, src_hbm, dst_hbm, out_hbm,
               src_v, dst_v, bufs, acc, sems, ssems, cid, sid, zero_wait=None):
    wid = cid * 16 + sid

    for p in range(_NPH):
        pbase = wid * _CPW + p * _CPP
        if gather:
            pltpu.sync_copy(src_hbm.at[pl.ds(pbase, _CPP)], src_v)
        pltpu.sync_copy(dst_hbm.at[pl.ds(pbase, _CPP)], dst_v)

        if p == 0 and zero_wait is not None:
            zero_wait()
            plsc.subcore_barrier()

        def _fire(j, b, pbase=pbase):
            if gather:
                pltpu.make_async_copy(
                    vals_hbm.at[src_v.at[j]], bufs[b], sems[b]).start()
            else:
                row0 = jnp.minimum(pbase + j, _NCHUNK - 1) * _CH
                pltpu.make_async_copy(
                    vals_hbm.at[pl.ds(row0, _CH)], bufs[b], sems[b]).start()

        def _wait(b):
            pltpu.make_async_copy(
                vals_hbm.at[pl.ds(0, _CH)], bufs[b], sems[b]).wait()

        def _scat_start(j, b):
            pltpu.async_copy(bufs[b], acc.at[dst_v.at[j]], ssems[b], add=True)

        def _scat_wait(b):
            pltpu.make_async_copy(bufs[b], acc.at[dst_v.at[0]],
                                  ssems[b]).wait()

        _fire(0, 0)

        def body(jj, carry):
            for t in range(_UNROLL):
                j = jj * _UNROLL + t
                b = t % _NBUF
                ob = (t + 1) % _NBUF
                _wait(b)
                _scat_start(j, b)

                @pl.when(j >= 1)
                def _():
                    _scat_wait(ob)

                @pl.when(j + 1 < _CPP)
                def _():
                    _fire(j + 1, ob)
            return carry

        lax.fori_loop(0, _CPP // _UNROLL, body, 0)
        _scat_wait((_CPP - 1) % _NBUF)

    plsc.subcore_barrier()
    pltpu.sync_copy(acc.at[pl.ds(sid * _ZOFF, _ZWIN)],
                    out_hbm.at[cid, pl.ds(sid * _ZOFF, _ZWIN)])


def _zero_acc_start(zeros_hbm, acc, sid, sems):
    plsc.subcore_barrier()
    cp = pltpu.make_async_copy(zeros_hbm.at[pl.ds(sid * _ZOFF, _ZWIN)],
                               acc.at[pl.ds(sid * _ZOFF, _ZWIN)], sems[0])
    cp.start()
    return cp.wait


_scratch = [
    pltpu.VMEM((_CPP, _CH), jnp.int32),
    pltpu.VMEM((_CPP, _CH), jnp.int32),
    [pltpu.VMEM((_CH, _D), jnp.float32) for _ in range(_NBUF)],
    pltpu.VMEM_SHARED((_ACC_ROWS, _D), jnp.float32),
    [pltpu.SemaphoreType.DMA for _ in range(_NBUF)],
    [pltpu.SemaphoreType.DMA for _ in range(_NBUF)],
]

_out_t = jax.ShapeDtypeStruct((2, _ACC_ROWS, _D), jnp.float32)


@functools.partial(pl.kernel, out_type=(_out_t, _out_t), mesh=_mesh,
                   scratch_types=_scratch)
def _sc_layer0(ea_hbm, x_hbm, src_hbm, dst_hbm, zeros_hbm, ae_out, y0_out,
               src_v, dst_v, bufs, acc, sems, ssems):
    cid = lax.axis_index("c")
    sid = lax.axis_index("s")
    zw = _zero_acc_start(zeros_hbm, acc, sid, sems)
    _emit_pass(False, ea_hbm, src_hbm, dst_hbm, ae_out,
               src_v, dst_v, bufs, acc, sems, ssems, cid, sid, zero_wait=zw)
    zw = _zero_acc_start(zeros_hbm, acc, sid, sems)
    _emit_pass(True, x_hbm, src_hbm, dst_hbm, y0_out,
               src_v, dst_v, bufs, acc, sems, ssems, cid, sid, zero_wait=zw)


@functools.partial(pl.kernel, out_type=_out_t, mesh=_mesh,
                   scratch_types=_scratch)
def _sc_gather_scatter(vals_hbm, src_hbm, dst_hbm, zeros_hbm, out_hbm,
                       src_v, dst_v, bufs, acc, sems, ssems):
    cid = lax.axis_index("c")
    sid = lax.axis_index("s")
    zw = _zero_acc_start(zeros_hbm, acc, sid, sems)
    _emit_pass(True, vals_hbm, src_hbm, dst_hbm, out_hbm,
               src_v, dst_v, bufs, acc, sems, ssems, cid, sid, zero_wait=zw)


def _mlp_body(ae_ref, y_ref, h_ref, w1a_ref, w1b_ref, w2_ref, b1_ref, b2_ref,
              o_ref, *, relu_out):
    ae = ae_ref[0] + ae_ref[1]
    y = y_ref[0] + y_ref[1] + h_ref[...]
    hid = (jnp.dot(ae, w1a_ref[...], preferred_element_type=jnp.float32)
           + jnp.dot(y, w1b_ref[...], preferred_element_type=jnp.float32)
           + b1_ref[...])
    hid = jnp.maximum(hid, 0.0)
    out = jnp.dot(hid, w2_ref[...], preferred_element_type=jnp.float32) + b2_ref[...]
    if relu_out:
        out = jnp.maximum(out, 0.0)
    o_ref[...] = out


def _mlp(ae2, y2, h, W1, b1, W2, b2, relu_out):
    B = 2000
    W1a = W1[:_D]
    W1b = W1[_D:]
    b1e = (b1 + W1[_D - 1]).reshape(1, 2 * _D)
    b2r = b2.reshape(1, _D)
    return pl.pallas_call(
        functools.partial(_mlp_body, relu_out=relu_out),
        grid=(_N // B,),
        in_specs=[
            pl.BlockSpec((2, B, _D), lambda i: (0, i, 0)),
            pl.BlockSpec((2, B, _D), lambda i: (0, i, 0)),
            pl.BlockSpec((B, _D), lambda i: (i, 0)),
            pl.BlockSpec((_D, 2 * _D), lambda i: (0, 0)),
            pl.BlockSpec((_D, 2 * _D), lambda i: (0, 0)),
            pl.BlockSpec((2 * _D, _D), lambda i: (0, 0)),
            pl.BlockSpec((1, 2 * _D), lambda i: (0, 0)),
            pl.BlockSpec((1, _D), lambda i: (0, 0)),
        ],
        out_specs=pl.BlockSpec((B, _D), lambda i: (i, 0)),
        out_shape=jax.ShapeDtypeStruct((_N, _D), jnp.float32),
    )(ae2, y2, h, W1a, W1b, W2, b1e, b2r)


def kernel(x, edge_index, edge_attr,
           W1_0, b1_0, W2_0, b2_0,
           W1_1, b1_1, W2_1, b2_1):
    src = edge_index[0]
    dst = edge_index[1]
    npad = _EPAD_ROWS * _CH - _E
    pad_src = jnp.arange(npad, dtype=jnp.int32) % _N
    srcP = jnp.concatenate([src, pad_src]).reshape(_EPAD_ROWS, _CH)
    dstP = jnp.concatenate([dst, jnp.full((npad,), _N, jnp.int32)]).reshape(_EPAD_ROWS, _CH)
    zeros_nd = jnp.zeros((_ACC_ROWS, _D), jnp.float32)

    ae2, y0 = _sc_layer0(edge_attr, x, srcP, dstP, zeros_nd)
    h1 = _mlp(ae2, y0, x, W1_0, b1_0, W2_0, b2_0, relu_out=True)
    y1 = _sc_gather_scatter(h1, srcP, dstP, zeros_nd)
    h2 = _mlp(ae2, y1, h1, W1_1, b1_1, W2_1, b2_1, relu_out=False)
    return h2


# speedup vs baseline: 1.1661x; 1.1661x over previous
import functools

import jax
import jax.numpy as jnp
from jax import lax
from jax.experimental import pallas as pl
from jax.experimental.pallas import tpu as pltpu
from jax.experimental.pallas import tpu_sc as plsc

_N = 10000
_E = 320000
_D = 128
_CH = 128
_NW = 32
_NCHUNK = _E // _CH
_CPW = 80
_EPAD_ROWS = _NW * _CPW
_ACC_ROWS = 10016
_ZOFF = 624
_ZWIN = 656
_NPH = 2
_CPP = _CPW // _NPH

_mesh = plsc.VectorSubcoreMesh(
    core_axis_name="c", subcore_axis_name="s", num_cores=2, num_subcores=16
)


_NBUF = 2


_UNROLL = 8


def _emit_pass(gather, vals_hbm, src_hbm, dst_hbm, out_hbm,
               src_v, dst_v, bufs, acc, sems, cid, sid, zero_wait=None):
    wid = cid * 16 + sid

    for p in range(_NPH):
        pbase = wid * _CPW + p * _CPP
        if gather:
            pltpu.sync_copy(src_hbm.at[pl.ds(pbase, _CPP)], src_v)
        pltpu.sync_copy(dst_hbm.at[pl.ds(pbase, _CPP)], dst_v)

        if p == 0 and zero_wait is not None:
            zero_wait()
            plsc.subcore_barrier()

        def _fire(j, b, pbase=pbase):
            if gather:
                pltpu.make_async_copy(
                    vals_hbm.at[src_v.at[j]], bufs[b], sems[b]).start()
            else:
                row0 = jnp.minimum(pbase + j, _NCHUNK - 1) * _CH
                pltpu.make_async_copy(
                    vals_hbm.at[pl.ds(row0, _CH)], bufs[b], sems[b]).start()

        def _wait(b):
            pltpu.make_async_copy(
                vals_hbm.at[pl.ds(0, _CH)], bufs[b], sems[b]).wait()

        for b in range(_NBUF):
            _fire(b, b)

        def body(jj, carry):
            for t in range(_UNROLL):
                j = jj * _UNROLL + t
                b = t % _NBUF
                _wait(b)
                pltpu.sync_copy(bufs[b], acc.at[dst_v.at[j]], add=True)

                @pl.when(j + _NBUF < _CPP)
                def _():
                    _fire(j + _NBUF, b)
            return carry

        lax.fori_loop(0, _CPP // _UNROLL, body, 0)

    plsc.subcore_barrier()
    pltpu.sync_copy(acc.at[pl.ds(sid * _ZOFF, _ZWIN)],
                    out_hbm.at[cid, pl.ds(sid * _ZOFF, _ZWIN)])


def _zero_acc_start(zeros_hbm, acc, sid, sems):
    plsc.subcore_barrier()
    cp = pltpu.make_async_copy(zeros_hbm.at[pl.ds(sid * _ZOFF, _ZWIN)],
                               acc.at[pl.ds(sid * _ZOFF, _ZWIN)], sems[0])
    cp.start()
    return cp.wait


_scratch = [
    pltpu.VMEM((_CPP, _CH), jnp.int32),
    pltpu.VMEM((_CPP, _CH), jnp.int32),
    [pltpu.VMEM((_CH, _D), jnp.float32) for _ in range(_NBUF)],
    pltpu.VMEM_SHARED((_ACC_ROWS, _D), jnp.float32),
    [pltpu.SemaphoreType.DMA for _ in range(_NBUF)],
]

_out_t = jax.ShapeDtypeStruct((2, _ACC_ROWS, _D), jnp.float32)


@functools.partial(pl.kernel, out_type=(_out_t, _out_t), mesh=_mesh,
                   scratch_types=_scratch)
def _sc_layer0(ea_hbm, x_hbm, src_hbm, dst_hbm, zeros_hbm, ae_out, y0_out,
               src_v, dst_v, bufs, acc, sems):
    cid = lax.axis_index("c")
    sid = lax.axis_index("s")
    zw = _zero_acc_start(zeros_hbm, acc, sid, sems)
    _emit_pass(False, ea_hbm, src_hbm, dst_hbm, ae_out,
               src_v, dst_v, bufs, acc, sems, cid, sid, zero_wait=zw)
    zw = _zero_acc_start(zeros_hbm, acc, sid, sems)
    _emit_pass(True, x_hbm, src_hbm, dst_hbm, y0_out,
               src_v, dst_v, bufs, acc, sems, cid, sid, zero_wait=zw)


@functools.partial(pl.kernel, out_type=_out_t, mesh=_mesh,
                   scratch_types=_scratch)
def _sc_gather_scatter(vals_hbm, src_hbm, dst_hbm, zeros_hbm, out_hbm,
                       src_v, dst_v, bufs, acc, sems):
    cid = lax.axis_index("c")
    sid = lax.axis_index("s")
    zw = _zero_acc_start(zeros_hbm, acc, sid, sems)
    _emit_pass(True, vals_hbm, src_hbm, dst_hbm, out_hbm,
               src_v, dst_v, bufs, acc, sems, cid, sid, zero_wait=zw)


def _mlp_body(ae_ref, y_ref, h_ref, w1a_ref, w1b_ref, w2_ref, b1_ref, b2_ref,
              o_ref, *, relu_out):
    ae = ae_ref[0] + ae_ref[1]
    y = y_ref[0] + y_ref[1] + h_ref[...]
    hid = (jnp.dot(ae, w1a_ref[...], preferred_element_type=jnp.float32)
           + jnp.dot(y, w1b_ref[...], preferred_element_type=jnp.float32)
           + b1_ref[...])
    hid = jnp.maximum(hid, 0.0)
    out = jnp.dot(hid, w2_ref[...], preferred_element_type=jnp.float32) + b2_ref[...]
    if relu_out:
        out = jnp.maximum(out, 0.0)
    o_ref[...] = out


def _mlp(ae2, y2, h, W1, b1, W2, b2, relu_out):
    B = 2000
    W1a = W1[:_D]
    W1b = W1[_D:]
    b1e = (b1 + W1[_D - 1]).reshape(1, 2 * _D)
    b2r = b2.reshape(1, _D)
    return pl.pallas_call(
        functools.partial(_mlp_body, relu_out=relu_out),
        grid=(_N // B,),
        in_specs=[
            pl.BlockSpec((2, B, _D), lambda i: (0, i, 0)),
            pl.BlockSpec((2, B, _D), lambda i: (0, i, 0)),
            pl.BlockSpec((B, _D), lambda i: (i, 0)),
            pl.BlockSpec((_D, 2 * _D), lambda i: (0, 0)),
            pl.BlockSpec((_D, 2 * _D), lambda i: (0, 0)),
            pl.BlockSpec((2 * _D, _D), lambda i: (0, 0)),
            pl.BlockSpec((1, 2 * _D), lambda i: (0, 0)),
            pl.BlockSpec((1, _D), lambda i: (0, 0)),
        ],
        out_specs=pl.BlockSpec((B, _D), lambda i: (i, 0)),
        out_shape=jax.ShapeDtypeStruct((_N, _D), jnp.float32),
    )(ae2, y2, h, W1a, W1b, W2, b1e, b2r)


def kernel(x, edge_index, edge_attr,
           W1_0, b1_0, W2_0, b2_0,
           W1_1, b1_1, W2_1, b2_1):
    src = edge_index[0]
    dst = edge_index[1]
    npad = _EPAD_ROWS * _CH - _E
    pad_src = jnp.arange(npad, dtype=jnp.int32) % _N
    srcP = jnp.concatenate([src, pad_src]).reshape(_EPAD_ROWS, _CH)
    dstP = jnp.concatenate([dst, jnp.full((npad,), _N, jnp.int32)]).reshape(_EPAD_ROWS, _CH)
    zeros_nd = jnp.zeros((_ACC_ROWS, _D), jnp.float32)

    ae2, y0 = _sc_layer0(edge_attr, x, srcP, dstP, zeros_nd)
    h1 = _mlp(ae2, y0, x, W1_0, b1_0, W2_0, b2_0, relu_out=True)
    y1 = _sc_gather_scatter(h1, srcP, dstP, zeros_nd)
    h2 = _mlp(ae2, y1, h1, W1_1, b1_1, W2_1, b2_1, relu_out=False)
    return h2
